# Initial kernel scaffold; baseline (speedup 1.0000x reference)
import jax
import jax.numpy as jnp
from jax.experimental import pallas as pl


def kernel(x, edge_index, W1a, b1a, W1b, b1b, W2a, b2a, W2b, b2b):
    raise NotImplementedError("write your pallas kernel here")


# speedup vs baseline: 3.2012x; 3.2012x over previous
import functools

import jax
import jax.numpy as jnp
from jax import lax
from jax.experimental import pallas as pl
from jax.experimental.pallas import tpu as pltpu
from jax.experimental.pallas import tpu_sc as plsc

N_NODES = 10000
N_EDGES = 320000
D = 128

NC = 2
NS = 16
NW = NC * NS

CSZ = 128
CH = 80
IB = 8
NBLK = CH // IB
EPT = CH * CSZ
E_PAD = NW * EPT

N_PAD = 10240
RPT = N_PAD // NS
RCH = RPT // CSZ
PAD_DST = N_NODES + 8

_sc_mesh = plsc.VectorSubcoreMesh(core_axis_name="c", subcore_axis_name="s")


@functools.partial(
    pl.kernel,
    out_type=jax.ShapeDtypeStruct((NC, N_PAD, D), jnp.float32),
    mesh=_sc_mesh,
    scratch_types=[
        pltpu.VMEM_SHARED((N_PAD, D), jnp.float32),
        pltpu.VMEM((IB, CSZ), jnp.int32),
        pltpu.VMEM((IB, CSZ), jnp.int32),
        pltpu.VMEM((CSZ, D), jnp.float32),
        pltpu.VMEM((CSZ, D), jnp.float32),
        pltpu.SemaphoreType.DMA,
        pltpu.SemaphoreType.DMA,
    ],
)
def _sc_aggregate(x_hbm, src_hbm, dst_hbm, out_hbm,
                  acc, src_v, dst_v, buf0, buf1, sem0, sem1):
    cid = lax.axis_index("c")
    sid = lax.axis_index("s")
    gid = cid * NS + sid
    r0 = sid * RPT

    for k in range(RCH):
        rows = pl.ds(r0 + k * CSZ, CSZ)
        pltpu.sync_copy(x_hbm.at[rows], buf0)
        pltpu.sync_copy(buf0, acc.at[rows])
    plsc.subcore_barrier()

    def body(ib, carry):
        pltpu.sync_copy(src_hbm.at[gid, pl.ds(ib * IB, IB)], src_v)
        pltpu.sync_copy(dst_hbm.at[gid, pl.ds(ib * IB, IB)], dst_v)
        for jj in range(IB // 2):
            a = 2 * jj
            b = a + 1
            ga = pltpu.async_copy(x_hbm.at[src_v.at[a]], buf0, sem0)
            gb = pltpu.async_copy(x_hbm.at[src_v.at[b]], buf1, sem1)
            ga.wait()
            pltpu.sync_copy(buf0, acc.at[dst_v.at[a]], add=True)
            gb.wait()
            pltpu.sync_copy(buf1, acc.at[dst_v.at[b]], add=True)
        return carry

    lax.fori_loop(0, NBLK, body, 0)
    plsc.subcore_barrier()

    for k in range(RCH):
        rows = pl.ds(r0 + k * CSZ, CSZ)
        pltpu.sync_copy(acc.at[rows], buf0)
        pltpu.sync_copy(buf0, out_hbm.at[cid, rows])


_ROWS_BLK = 1024


def _mlp_body(final_relu, x_ref, p0_ref, p1_ref, wa_ref, ba_ref, wb_ref,
              bb_ref, o_ref):
    h = p0_ref[...] + p1_ref[...] - x_ref[...]
    h = jnp.dot(h, wa_ref[...], preferred_element_type=jnp.float32)
    h = jnp.maximum(h + ba_ref[...], 0.0)
    o = jnp.dot(h, wb_ref[...], preferred_element_type=jnp.float32)
    o = o + bb_ref[...]
    if final_relu:
        o = jnp.maximum(o, 0.0)
    o_ref[...] = o


def _mlp(x_pad, p0, p1, wa, ba, wb, bb, final_relu):
    row_spec = pl.BlockSpec((_ROWS_BLK, D), lambda i: (i, 0))
    full_spec = pl.BlockSpec((D, D), lambda i: (0, 0))
    bias_spec = pl.BlockSpec((1, D), lambda i: (0, 0))
    return pl.pallas_call(
        functools.partial(_mlp_body, final_relu),
        grid=(N_PAD // _ROWS_BLK,),
        in_specs=[row_spec, row_spec, row_spec,
                  full_spec, bias_spec, full_spec, bias_spec],
        out_specs=row_spec,
        out_shape=jax.ShapeDtypeStruct((N_PAD, D), jnp.float32),
    )(x_pad, p0, p1, wa, ba.reshape(1, D), wb, bb.reshape(1, D))


def kernel(x, edge_index, W1a, b1a, W1b, b1b, W2a, b2a, W2b, b2b):
    src = edge_index[0].astype(jnp.int32)
    dst = edge_index[1].astype(jnp.int32)
    pad_e = E_PAD - N_EDGES
    src_r = jnp.concatenate([src, jnp.zeros((pad_e,), jnp.int32)])
    src_r = src_r.reshape(NW, CH, CSZ)
    dst_r = jnp.concatenate([dst, jnp.full((pad_e,), PAD_DST, jnp.int32)])
    dst_r = dst_r.reshape(NW, CH, CSZ)
    x_pad = jnp.concatenate(
        [x, jnp.zeros((N_PAD - N_NODES, D), jnp.float32)])

    parts1 = _sc_aggregate(x_pad, src_r, dst_r)
    h1 = _mlp(x_pad, parts1[0], parts1[1], W1a, b1a, W1b, b1b,
              final_relu=True)
    parts2 = _sc_aggregate(h1, src_r, dst_r)
    out = _mlp(h1, parts2[0], parts2[1], W2a, b2a, W2b, b2b,
               final_relu=False)
    return out[:N_NODES]
